# baseline (speedup 1.0000x reference)
import functools

import jax
import jax.numpy as jnp
from jax import lax
from jax.experimental import pallas as pl
from jax.experimental.pallas import tpu as pltpu
from jax.experimental.pallas import tpu_sc as plsc

N_INTER = 2097152
N_COLS_ = 4
N_DELTA = 4096

NC = 2
NS = 16
NW = NC * NS
ROWS_W = N_INTER // NW
CROWS = 1024
NCHUNK = ROWS_W // CROWS
LANES = 16
VECS = CROWS * N_COLS_ // LANES
NBUF = 4
NGROUP = NCHUNK // NBUF

_mesh = plsc.VectorSubcoreMesh(core_axis_name="c", subcore_axis_name="s")


@functools.partial(
    pl.kernel,
    mesh=_mesh,
    out_type=jax.ShapeDtypeStruct((N_INTER, N_COLS_), jnp.float32),
    compiler_params=pltpu.CompilerParams(
        needs_layout_passes=False, use_tc_tiling_on_sc=False
    ),
    scratch_types=[
        pltpu.VMEM((N_DELTA,), jnp.float32),
        [pltpu.VMEM((CROWS, N_COLS_), jnp.int32)] * NBUF,
        [pltpu.VMEM((CROWS, N_COLS_), jnp.float32)] * NBUF,
        [pltpu.VMEM((CROWS, N_COLS_), jnp.float32)] * NBUF,
        [pltpu.SemaphoreType.DMA] * NBUF,
        [pltpu.SemaphoreType.DMA] * NBUF,
    ],
)
def _sc_gather_add(
    hp_hbm, idx_hbm, delta_hbm, out_hbm, delta_v, idx_v, val_v, res_v,
    sems_in, sems_out,
):
    wid = lax.axis_index("s") * NC + lax.axis_index("c")
    base = wid * ROWS_W
    pltpu.sync_copy(delta_hbm, delta_v)

    lanes = lax.iota(jnp.int32, LANES)
    rpat = lax.shift_right_logical(lanes, 2)
    cpat = lax.bitwise_and(lanes, 3)

    def start_in(g, b):
        off = base + g * CROWS
        pltpu.async_copy(idx_hbm.at[pl.ds(off, CROWS)], idx_v[b], sems_in[b])
        pltpu.async_copy(hp_hbm.at[pl.ds(off, CROWS)], val_v[b], sems_in[b])

    def wait_in(b):
        pltpu.make_async_copy(idx_hbm.at[pl.ds(base, CROWS)], idx_v[b], sems_in[b]).wait()
        pltpu.make_async_copy(hp_hbm.at[pl.ds(base, CROWS)], val_v[b], sems_in[b]).wait()

    def start_out(g, b):
        off = base + g * CROWS
        pltpu.async_copy(res_v[b], out_hbm.at[pl.ds(off, CROWS)], sems_out[b])

    def wait_out(b):
        pltpu.make_async_copy(res_v[b], out_hbm.at[pl.ds(base, CROWS)], sems_out[b]).wait()

    for b in range(NBUF):
        start_in(b, b)

    def group_body(G, carry):
        g0 = G * NBUF
        for b in range(NBUF):
            g = g0 + b
            wait_in(b)

            @pl.when(G > 0)
            def _():
                wait_out(b)

            ib, vb, rb = idx_v[b], val_v[b], res_v[b]

            @plsc.parallel_loop(0, VECS, unroll=8)
            def vec_body(i):
                rows = rpat + i * (LANES // N_COLS_)
                iv = plsc.load_gather(ib, [rows, cpat])
                vv = plsc.load_gather(vb, [rows, cpat])
                gv = plsc.load_gather(delta_v, [iv])
                plsc.store_scatter(rb, [rows, cpat], vv + gv)

            start_out(g, b)

            @pl.when(g + NBUF < NCHUNK)
            def _():
                start_in(g + NBUF, b)
        return carry

    lax.fori_loop(0, NGROUP, group_body, 0)
    for b in range(NBUF):
        wait_out(b)


def kernel(handler_parameters, handler_parameter_idx, parameter_delta):
    return _sc_gather_add(handler_parameters, handler_parameter_idx, parameter_delta)


# speedup vs baseline: 33.4778x; 33.4778x over previous
import functools

import jax
import jax.numpy as jnp
from jax import lax
from jax.experimental import pallas as pl
from jax.experimental.pallas import tpu as pltpu
from jax.experimental.pallas import tpu_sc as plsc

N_INTER = 2097152
N_COLS_ = 4
N_DELTA = 4096

NC = 2
NS = 16
NW = NC * NS
PER_W = N_INTER // NW
CHUNK = 8192
NCHUNK = PER_W // CHUNK
LANES = 16
VECS = CHUNK // LANES
NBUF = 2
NGROUP = NCHUNK // NBUF

_mesh = plsc.VectorSubcoreMesh(core_axis_name="c", subcore_axis_name="s")


@functools.partial(
    pl.kernel,
    mesh=_mesh,
    out_type=[jax.ShapeDtypeStruct((N_INTER,), jnp.float32)] * N_COLS_,
    compiler_params=pltpu.CompilerParams(needs_layout_passes=False),
    scratch_types=[
        pltpu.VMEM((N_DELTA,), jnp.float32),
        [pltpu.VMEM((CHUNK,), jnp.int32)] * NBUF,
        [pltpu.VMEM((CHUNK,), jnp.float32)] * NBUF,
        [pltpu.VMEM((CHUNK,), jnp.float32)] * NBUF,
        [pltpu.SemaphoreType.DMA] * NBUF,
        [pltpu.SemaphoreType.DMA] * NBUF,
    ],
)
def _sc_gather_add(
    hp0, hp1, hp2, hp3, idx0, idx1, idx2, idx3, delta_hbm,
    out0, out1, out2, out3,
    delta_v, idx_v, val_v, res_v, sems_in, sems_out,
):
    wid = lax.axis_index("s") * NC + lax.axis_index("c")
    base = wid * PER_W
    pltpu.sync_copy(delta_hbm, delta_v)

    for hp_hbm, idx_hbm, out_hbm in (
        (hp0, idx0, out0),
        (hp1, idx1, out1),
        (hp2, idx2, out2),
        (hp3, idx3, out3),
    ):
        def start_in(g, b):
            off = base + g * CHUNK
            pltpu.async_copy(idx_hbm.at[pl.ds(off, CHUNK)], idx_v[b], sems_in[b])
            pltpu.async_copy(hp_hbm.at[pl.ds(off, CHUNK)], val_v[b], sems_in[b])

        def wait_in(b):
            pltpu.make_async_copy(idx_hbm.at[pl.ds(base, CHUNK)], idx_v[b], sems_in[b]).wait()
            pltpu.make_async_copy(hp_hbm.at[pl.ds(base, CHUNK)], val_v[b], sems_in[b]).wait()

        def start_out(g, b):
            off = base + g * CHUNK
            pltpu.async_copy(res_v[b], out_hbm.at[pl.ds(off, CHUNK)], sems_out[b])

        def wait_out(b):
            pltpu.make_async_copy(res_v[b], out_hbm.at[pl.ds(base, CHUNK)], sems_out[b]).wait()

        for b in range(NBUF):
            start_in(b, b)

        def group_body(G, carry):
            g0 = G * NBUF
            for b in range(NBUF):
                g = g0 + b
                wait_in(b)

                @pl.when(G > 0)
                def _():
                    wait_out(b)

                ib, vb, rb = idx_v[b], val_v[b], res_v[b]

                @plsc.parallel_loop(0, VECS, unroll=8)
                def vec_body(i):
                    s = pl.ds(i * LANES, LANES)
                    rb[s] = vb[s] + plsc.load_gather(delta_v, [ib[s]])

                start_out(g, b)

                @pl.when(g + NBUF < NCHUNK)
                def _():
                    start_in(g + NBUF, b)
            return carry

        lax.fori_loop(0, NGROUP, group_body, 0)
        for b in range(NBUF):
            wait_out(b)


def kernel(handler_parameters, handler_parameter_idx, parameter_delta):
    hp_cols = [handler_parameters[:, c] for c in range(N_COLS_)]
    idx_cols = [handler_parameter_idx[:, c] for c in range(N_COLS_)]
    outs = _sc_gather_add(*hp_cols, *idx_cols, parameter_delta)
    return jnp.stack(outs, axis=1)
